# baseline (speedup 1.0000x reference)
import functools

import jax
import jax.numpy as jnp
from jax import lax
from jax.experimental import pallas as pl
from jax.experimental.pallas import tpu as pltpu
from jax.experimental.pallas import tpu_sc as plsc

SHAPE = (4, 16, 4096, 128)
N = SHAPE[0] * SHAPE[1] * SHAPE[2] * SHAPE[3]
KEEP = int(N * (1.0 - 0.5))
M_DROP = N - KEEP

NC, NS = 2, 16
NW = NC * NS
PER_W = N // NW
CH = 16384
NCH = PER_W // CH
UNROLL = 16

NB1 = 1 << 16
NB2 = 1 << 15

@functools.cache
def _sc_mesh():
    return plsc.VectorSubcoreMesh(
        core_axis_name="c", subcore_axis_name="s",
        num_cores=NC, num_subcores=NS,
    )


def _zero_i32(ref, nwords):
    zeros16 = jnp.zeros((16,), jnp.int32)

    def zbody(i, carry):
        for u in range(UNROLL):
            ref[pl.ds((i * UNROLL + u) * 16, 16)] = zeros16
        return carry

    lax.fori_loop(0, nwords // (16 * UNROLL), zbody, 0)


@functools.cache
def _hist1_kernel():
    return pl.kernel(
        _hist1_body,
        out_type=jax.ShapeDtypeStruct((NW, NB1), jnp.int32),
        mesh=_sc_mesh(),
        compiler_params=pltpu.CompilerParams(needs_layout_passes=False),
        scratch_types=[
            pltpu.VMEM((CH,), jnp.float32),
            pltpu.VMEM((CH,), jnp.float32),
            pltpu.VMEM((NB1,), jnp.int32),
            pltpu.SemaphoreType.DMA,
            pltpu.SemaphoreType.DMA,
        ],
    )


def _hist1_body(x_hbm, out_hbm, buf0, buf1, hist, sem0, sem1):
    wid = lax.axis_index("s") * NC + lax.axis_index("c")
    base = wid * PER_W
    _zero_i32(hist, NB1)

    def consume(buf):
        @plsc.parallel_loop(0, CH // 16, unroll=UNROLL)
        def _inner(j):
            v = buf[pl.ds(j * 16, 16)]
            b = plsc.bitcast(v, jnp.int32)
            a = b & jnp.int32(0x7FFFFFFF)
            bn = a >> 15
            plsc.addupdate_scatter(hist, [bn], jnp.ones((16,), jnp.int32))

    pltpu.async_copy(x_hbm.at[pl.ds(base, CH)], buf0, sem0)

    def body(i, carry):
        off = base + 2 * i * CH
        pltpu.async_copy(x_hbm.at[pl.ds(off + CH, CH)], buf1, sem1)
        pltpu.make_async_copy(x_hbm.at[pl.ds(off, CH)], buf0, sem0).wait()
        consume(buf0)

        @pl.when(i + 1 < NCH // 2)
        def _():
            pltpu.async_copy(x_hbm.at[pl.ds(off + 2 * CH, CH)], buf0, sem0)

        pltpu.make_async_copy(x_hbm.at[pl.ds(off + CH, CH)], buf1, sem1).wait()
        consume(buf1)
        return carry

    lax.fori_loop(0, NCH // 2, body, 0)
    pltpu.sync_copy(hist, out_hbm.at[wid])


@functools.cache
def _hist2_kernel():
    return pl.kernel(
        _hist2_body,
        out_type=jax.ShapeDtypeStruct((NW, NB2), jnp.int32),
        mesh=_sc_mesh(),
        compiler_params=pltpu.CompilerParams(needs_layout_passes=False),
        scratch_types=[
            pltpu.VMEM((CH,), jnp.float32),
            pltpu.VMEM((CH,), jnp.float32),
            pltpu.VMEM((NB2,), jnp.int32),
            pltpu.VMEM((16,), jnp.int32),
            pltpu.SemaphoreType.DMA,
            pltpu.SemaphoreType.DMA,
        ],
    )


def _hist2_body(x_hbm, bsel_hbm, out_hbm, buf0, buf1, hist, bvmem, sem0, sem1):
    wid = lax.axis_index("s") * NC + lax.axis_index("c")
    base = wid * PER_W
    _zero_i32(hist, NB2)
    pltpu.sync_copy(bsel_hbm, bvmem)
    bvec = bvmem[...]

    def consume(buf):
        @plsc.parallel_loop(0, CH // 16, unroll=UNROLL)
        def _inner(j):
            v = buf[pl.ds(j * 16, 16)]
            b = plsc.bitcast(v, jnp.int32)
            a = b & jnp.int32(0x7FFFFFFF)
            bn = a >> 15
            match = bn == bvec
            low = a & jnp.int32(0x7FFF)
            plsc.addupdate_scatter(hist, [low], jnp.ones((16,), jnp.int32),
                                   mask=match)

    pltpu.async_copy(x_hbm.at[pl.ds(base, CH)], buf0, sem0)

    def body(i, carry):
        off = base + 2 * i * CH
        pltpu.async_copy(x_hbm.at[pl.ds(off + CH, CH)], buf1, sem1)
        pltpu.make_async_copy(x_hbm.at[pl.ds(off, CH)], buf0, sem0).wait()
        consume(buf0)

        @pl.when(i + 1 < NCH // 2)
        def _():
            pltpu.async_copy(x_hbm.at[pl.ds(off + 2 * CH, CH)], buf0, sem0)

        pltpu.make_async_copy(x_hbm.at[pl.ds(off + CH, CH)], buf1, sem1).wait()
        consume(buf1)
        return carry

    lax.fori_loop(0, NCH // 2, body, 0)
    pltpu.sync_copy(hist, out_hbm.at[wid])


def _iscan(x, axis):
    n = x.shape[axis]
    s = 1
    while s < n:
        pad = jnp.zeros_like(lax.slice_in_dim(x, 0, s, axis=axis))
        shifted = jnp.concatenate(
            [pad, lax.slice_in_dim(x, 0, n - s, axis=axis)], axis=axis)
        x = x + shifted
        s *= 2
    return x


def _select_bin(ht, m_drop):
    ccol = _iscan(ht, 1)
    rt = ccol[:, 127:128]
    rowc = _iscan(rt, 0)
    incl = ccol + (rowc - rt)
    sel = (incl > m_drop) & ((incl - ht) <= m_drop)
    fidx = (lax.broadcasted_iota(jnp.int32, ht.shape, 0) * 128
            + lax.broadcasted_iota(jnp.int32, ht.shape, 1))
    bstar = jnp.sum(jnp.where(sel, fidx, 0))
    total = rowc[-1, 0]
    gstar = jnp.sum(jnp.where(sel, total - incl, 0))
    return bstar, gstar


def _scan1_body(hs_ref, b_ref, r_ref):
    ht = jnp.sum(hs_ref[...], axis=0)
    bstar, gstar = _select_bin(ht, jnp.int32(M_DROP))
    rstar = jnp.int32(KEEP) - gstar
    b_ref[...] = jnp.full((16,), bstar, jnp.int32)
    r_ref[...] = jnp.full((16,), rstar, jnp.int32)


def _scan2_body(hs_ref, b_ref, r_ref, t_ref):
    ht2 = jnp.sum(hs_ref[...], axis=0)
    n2 = jnp.sum(ht2)
    rstar = r_ref[...][0]
    jstar, _ = _select_bin(ht2, n2 - rstar)
    bstar = b_ref[...][0]
    tbits = (bstar << 15) | jstar
    t_ref[...] = lax.bitcast_convert_type(
        jnp.full((8, 128), tbits, jnp.int32), jnp.float32)


def _mask_body(t_ref, x_ref, o_ref):
    t = t_ref[0, 0]
    x = x_ref[...]
    o_ref[...] = jnp.where(jnp.abs(x) >= t, x, jnp.float32(0.0))


MASK_BLK = 16384


def kernel(kv_cache):
    x = kv_cache.reshape(-1)
    hists1 = _hist1_kernel()(x)
    b16, r16 = pl.pallas_call(
        _scan1_body,
        out_shape=[jax.ShapeDtypeStruct((16,), jnp.int32),
                   jax.ShapeDtypeStruct((16,), jnp.int32)],
    )(hists1.reshape(NW, NB1 // 128, 128))
    hists2 = _hist2_kernel()(x, b16)
    t = pl.pallas_call(
        _scan2_body,
        out_shape=jax.ShapeDtypeStruct((8, 128), jnp.float32),
    )(hists2.reshape(NW, NB2 // 128, 128), b16, r16)
    x2 = x.reshape(N // 128, 128)
    y = pl.pallas_call(
        _mask_body,
        grid=(N // 128 // MASK_BLK,),
        in_specs=[pl.BlockSpec((8, 128), lambda i: (0, 0)),
                  pl.BlockSpec((MASK_BLK, 128), lambda i: (i, 0))],
        out_specs=pl.BlockSpec((MASK_BLK, 128), lambda i: (i, 0)),
        out_shape=jax.ShapeDtypeStruct(x2.shape, x2.dtype),
    )(t, x2)
    return y.reshape(kv_cache.shape)


# speedup vs baseline: 1.1677x; 1.1677x over previous
import functools

import jax
import jax.numpy as jnp
from jax import lax
from jax.experimental import pallas as pl
from jax.experimental.pallas import tpu as pltpu
from jax.experimental.pallas import tpu_sc as plsc

SHAPE = (4, 16, 4096, 128)
N = SHAPE[0] * SHAPE[1] * SHAPE[2] * SHAPE[3]
KEEP = int(N * (1.0 - 0.5))
M_DROP = N - KEEP

NC, NS = 2, 16
NW = NC * NS
PER_W = N // NW
CH = 16384
NCH = PER_W // CH
UNROLL = 8

NB1 = 1 << 16
NB2 = 1 << 15

@functools.cache
def _sc_mesh():
    return plsc.VectorSubcoreMesh(
        core_axis_name="c", subcore_axis_name="s",
        num_cores=NC, num_subcores=NS,
    )


def _zero_i32(ref, nwords):
    zeros16 = jnp.zeros((16,), jnp.int32)

    def zbody(i, carry):
        for u in range(UNROLL):
            ref[pl.ds((i * UNROLL + u) * 16, 16)] = zeros16
        return carry

    lax.fori_loop(0, nwords // (16 * UNROLL), zbody, 0)


@functools.cache
def _hist1_kernel():
    return pl.kernel(
        _hist1_body,
        out_type=jax.ShapeDtypeStruct((NW, NB1), jnp.int32),
        mesh=_sc_mesh(),
        compiler_params=pltpu.CompilerParams(needs_layout_passes=False),
        scratch_types=[
            pltpu.VMEM((CH,), jnp.float32),
            pltpu.VMEM((CH,), jnp.float32),
            pltpu.VMEM((NB1,), jnp.int32),
            pltpu.SemaphoreType.DMA,
            pltpu.SemaphoreType.DMA,
        ],
    )


def _hist1_body(x_hbm, out_hbm, buf0, buf1, hist, sem0, sem1):
    wid = lax.axis_index("s") * NC + lax.axis_index("c")
    base = wid * PER_W
    _zero_i32(hist, NB1)

    def consume(buf):
        @plsc.parallel_loop(0, CH // 16, unroll=UNROLL)
        def _inner(j):
            v = buf[pl.ds(j * 16, 16)]
            b = plsc.bitcast(v, jnp.int32)
            a = b & jnp.int32(0x7FFFFFFF)
            bn = a >> 15
            plsc.addupdate_scatter(hist, [bn], jnp.ones((16,), jnp.int32))

    pltpu.async_copy(x_hbm.at[pl.ds(base, CH)], buf0, sem0)

    def body(i, carry):
        off = base + 2 * i * CH
        pltpu.async_copy(x_hbm.at[pl.ds(off + CH, CH)], buf1, sem1)
        pltpu.make_async_copy(x_hbm.at[pl.ds(off, CH)], buf0, sem0).wait()
        consume(buf0)

        @pl.when(i + 1 < NCH // 2)
        def _():
            pltpu.async_copy(x_hbm.at[pl.ds(off + 2 * CH, CH)], buf0, sem0)

        pltpu.make_async_copy(x_hbm.at[pl.ds(off + CH, CH)], buf1, sem1).wait()
        consume(buf1)
        return carry

    lax.fori_loop(0, NCH // 2, body, 0)
    pltpu.sync_copy(hist, out_hbm.at[wid])


@functools.cache
def _hist2_kernel():
    return pl.kernel(
        _hist2_body,
        out_type=jax.ShapeDtypeStruct((NW, NB2), jnp.int32),
        mesh=_sc_mesh(),
        compiler_params=pltpu.CompilerParams(needs_layout_passes=False),
        scratch_types=[
            pltpu.VMEM((CH,), jnp.float32),
            pltpu.VMEM((CH,), jnp.float32),
            pltpu.VMEM((NB2,), jnp.int32),
            pltpu.VMEM((16,), jnp.int32),
            pltpu.SemaphoreType.DMA,
            pltpu.SemaphoreType.DMA,
        ],
    )


def _hist2_body(x_hbm, bsel_hbm, out_hbm, buf0, buf1, hist, bvmem, sem0, sem1):
    wid = lax.axis_index("s") * NC + lax.axis_index("c")
    base = wid * PER_W
    _zero_i32(hist, NB2)
    pltpu.sync_copy(bsel_hbm, bvmem)
    bvec = bvmem[...]

    def consume(buf):
        @plsc.parallel_loop(0, CH // 16, unroll=UNROLL)
        def _inner(j):
            v = buf[pl.ds(j * 16, 16)]
            b = plsc.bitcast(v, jnp.int32)
            a = b & jnp.int32(0x7FFFFFFF)
            bn = a >> 15
            match = bn == bvec
            low = a & jnp.int32(0x7FFF)
            plsc.addupdate_scatter(hist, [low], jnp.ones((16,), jnp.int32),
                                   mask=match)

    pltpu.async_copy(x_hbm.at[pl.ds(base, CH)], buf0, sem0)

    def body(i, carry):
        off = base + 2 * i * CH
        pltpu.async_copy(x_hbm.at[pl.ds(off + CH, CH)], buf1, sem1)
        pltpu.make_async_copy(x_hbm.at[pl.ds(off, CH)], buf0, sem0).wait()
        consume(buf0)

        @pl.when(i + 1 < NCH // 2)
        def _():
            pltpu.async_copy(x_hbm.at[pl.ds(off + 2 * CH, CH)], buf0, sem0)

        pltpu.make_async_copy(x_hbm.at[pl.ds(off + CH, CH)], buf1, sem1).wait()
        consume(buf1)
        return carry

    lax.fori_loop(0, NCH // 2, body, 0)
    pltpu.sync_copy(hist, out_hbm.at[wid])


def _iscan(x, axis):
    n = x.shape[axis]
    s = 1
    while s < n:
        pad = jnp.zeros_like(lax.slice_in_dim(x, 0, s, axis=axis))
        shifted = jnp.concatenate(
            [pad, lax.slice_in_dim(x, 0, n - s, axis=axis)], axis=axis)
        x = x + shifted
        s *= 2
    return x


def _select_bin(ht, m_drop):
    ccol = _iscan(ht, 1)
    rt = ccol[:, 127:128]
    rowc = _iscan(rt, 0)
    incl = ccol + (rowc - rt)
    sel = (incl > m_drop) & ((incl - ht) <= m_drop)
    fidx = (lax.broadcasted_iota(jnp.int32, ht.shape, 0) * 128
            + lax.broadcasted_iota(jnp.int32, ht.shape, 1))
    bstar = jnp.sum(jnp.where(sel, fidx, 0))
    total = rowc[-1, 0]
    gstar = jnp.sum(jnp.where(sel, total - incl, 0))
    return bstar, gstar


def _scan1_body(hs_ref, b_ref, r_ref):
    ht = jnp.sum(hs_ref[...], axis=0).reshape(NB1 // 128, 128)
    bstar, gstar = _select_bin(ht, jnp.int32(M_DROP))
    rstar = jnp.int32(KEEP) - gstar
    b_ref[...] = jnp.full((16,), bstar, jnp.int32)
    r_ref[...] = jnp.full((16,), rstar, jnp.int32)


def _scan2_body(hs_ref, b_ref, r_ref, t_ref):
    ht2 = jnp.sum(hs_ref[...], axis=0).reshape(NB2 // 128, 128)
    n2 = jnp.sum(ht2)
    rstar = r_ref[...][0]
    jstar, _ = _select_bin(ht2, n2 - rstar)
    bstar = b_ref[...][0]
    tbits = (bstar << 15) | jstar
    t_ref[...] = lax.bitcast_convert_type(
        jnp.full((8, 128), tbits, jnp.int32), jnp.float32)


def _mask_body(t_ref, x_ref, o_ref):
    t = t_ref[0, 0]
    x = x_ref[...]
    o_ref[...] = jnp.where(jnp.abs(x) >= t, x, jnp.float32(0.0))


MASK_BLK = 16384


def kernel(kv_cache):
    x = kv_cache.reshape(-1)
    hists1 = _hist1_kernel()(x)
    b16, r16 = pl.pallas_call(
        _scan1_body,
        out_shape=[jax.ShapeDtypeStruct((16,), jnp.int32),
                   jax.ShapeDtypeStruct((16,), jnp.int32)],
    )(hists1)
    hists2 = _hist2_kernel()(x, b16)
    t = pl.pallas_call(
        _scan2_body,
        out_shape=jax.ShapeDtypeStruct((8, 128), jnp.float32),
    )(hists2, b16, r16)
    x2 = x.reshape(N // 128, 128)
    y = pl.pallas_call(
        _mask_body,
        grid=(N // 128 // MASK_BLK,),
        in_specs=[pl.BlockSpec((8, 128), lambda i: (0, 0)),
                  pl.BlockSpec((MASK_BLK, 128), lambda i: (i, 0))],
        out_specs=pl.BlockSpec((MASK_BLK, 128), lambda i: (i, 0)),
        out_shape=jax.ShapeDtypeStruct(x2.shape, x2.dtype),
    )(t, x2)
    return y.reshape(kv_cache.shape)
